# baseline (speedup 1.0000x reference)
import dataclasses
import functools

import jax
import jax.numpy as jnp
from jax import lax
from jax.experimental import pallas as pl
from jax.experimental.pallas import tpu as pltpu
from jax.experimental.pallas import tpu_sc as plsc

LANES = 16
NUM_WORKERS = 32
CHUNK = 128
REL_ROWS = 64


def _vrsqrt(s, steps=3):
    i = lax.bitcast_convert_type(s, jnp.int32)
    i = jnp.int32(0x5F3759DF) - (i >> 1)
    y = lax.bitcast_convert_type(i, jnp.float32)
    h = s * 0.5
    for _ in range(steps):
        y = y * (1.5 - h * y * y)
    return y


def _sumsq_tree(vs):
    sq = [v * v for v in vs]
    while len(sq) > 1:
        sq = [a + b for a, b in zip(sq[::2], sq[1::2])]
    return sq[0]


def _row_scale(buf, r, nvec, steps=3):
    vs = [buf[r, pl.ds(k * LANES, LANES)] for k in range(nvec)]
    s = jnp.maximum(jnp.sum(_sumsq_tree(vs)), 1e-12)
    y = _vrsqrt(jnp.broadcast_to(s, (LANES,)), steps)
    return vs, y


def _transe_body(nch, nrel, ent_hbm, rel_hbm, idxe_hbm, idxr_hbm, out_hbm,
                 idxe_v, idxr_v, reln_sp, bufe0, bufe1, bufe2,
                 bufr0, bufr1, bufo0, bufo1,
                 seme0, seme1, seme2, semr0, semr1, semo0, semo1):
    d = ent_hbm.shape[1]
    nvec = d // LANES
    sid = lax.axis_index("s")
    wid = sid * 2 + lax.axis_index("c")
    base = wid * (nch * CHUNK)

    bufe = [bufe0, bufe1, bufe2]
    bufr = [bufr0, bufr1]
    bufo = [bufo0, bufo1]
    seme = [seme0, seme1, seme2]
    semr = [semr0, semr1]
    semo = [semo0, semo1]
    def ent_gather(j, slot):
        return pltpu.async_copy(ent_hbm.at[idxe_v.at[j]], bufe[slot],
                                seme[slot])

    pltpu.sync_copy(idxe_hbm.at[wid], idxe_v)
    pltpu.sync_copy(idxr_hbm.at[wid], idxr_v)

    entc = [ent_gather(0, 0), ent_gather(1, 1), None]

    start = jnp.minimum(sid * REL_ROWS, nrel - REL_ROWS)
    pltpu.sync_copy(rel_hbm.at[pl.ds(start, REL_ROWS)],
                    bufe2.at[pl.ds(0, REL_ROWS)])

    @pl.loop(0, REL_ROWS)
    def _(r):
        vs, y = _row_scale(bufe2, r, nvec)
        for k in range(nvec):
            bufe2[r, pl.ds(k * LANES, LANES)] = vs[k] * y

    pltpu.sync_copy(bufe2.at[pl.ds(0, REL_ROWS)],
                    reln_sp.at[pl.ds(start, REL_ROWS)])
    plsc.subcore_barrier()

    relc = [
        pltpu.async_copy(reln_sp.at[idxr_v.at[0]], bufr[0], semr[0]),
        pltpu.async_copy(reln_sp.at[idxr_v.at[1]], bufr[1], semr[1]),
    ]
    outc = [None, None]

    for j in range(nch):
        cur = j % 2
        ecur = j % 3
        entc[ecur].wait()
        if j + 2 < nch:
            entc[(j + 2) % 3] = ent_gather(j + 2, (j + 2) % 3)
        relc[cur].wait()
        if outc[cur] is not None:
            outc[cur].wait()
        be, br, bo = bufe[ecur], bufr[cur], bufo[cur]

        @pl.loop(0, CHUNK, step=4)
        def _(r0):
            for r in (r0, r0 + 1, r0 + 2, r0 + 3):
                evs, ye = _row_scale(be, r, nvec, steps=2)
                for k in range(nvec):
                    bo[r, pl.ds(k * LANES, LANES)] = (
                        evs[k] * ye + br[r, pl.ds(k * LANES, LANES)])

        outc[cur] = pltpu.async_copy(
            bo, out_hbm.at[pl.ds(base + j * CHUNK, CHUNK)], semo[cur])
        if j + 2 < nch:
            relc[cur] = pltpu.async_copy(
                reln_sp.at[idxr_v.at[j + 2]], br, semr[cur])

    for cp in outc:
        if cp is not None:
            cp.wait()


def kernel(batch_source, batch_r, entity_embeddings, relation_embeddings):
    b = batch_source.shape[0]
    d = entity_embeddings.shape[1]
    nrel = relation_embeddings.shape[0]
    nch = b // (NUM_WORKERS * CHUNK)
    idx_e = batch_source.astype(jnp.int32).reshape(NUM_WORKERS, nch, CHUNK)
    idx_r = batch_r.astype(jnp.int32).reshape(NUM_WORKERS, nch, CHUNK)

    mesh = plsc.VectorSubcoreMesh(core_axis_name="c", subcore_axis_name="s")
    cp = pltpu.CompilerParams()
    if "needs_layout_passes" in pltpu.CompilerParams.__dataclass_fields__:
        cp = dataclasses.replace(cp, needs_layout_passes=False)
    run = pl.kernel(
        functools.partial(_transe_body, nch, nrel),
        out_type=jax.ShapeDtypeStruct((b, d), jnp.float32),
        mesh=mesh,
        scratch_types=(
            [pltpu.VMEM((nch, CHUNK), jnp.int32)] * 2
            + [pltpu.VMEM_SHARED((nrel, d), jnp.float32)]
            + [pltpu.VMEM((CHUNK, d), jnp.float32)] * 7
            + [pltpu.SemaphoreType.DMA] * 7
        ),
        compiler_params=cp,
    )
    return run(entity_embeddings, relation_embeddings, idx_e, idx_r)


# speedup vs baseline: 1.4572x; 1.4572x over previous
import dataclasses
import functools

import jax
import jax.numpy as jnp
from jax import lax
from jax.experimental import pallas as pl
from jax.experimental.pallas import tpu as pltpu
from jax.experimental.pallas import tpu_sc as plsc

LANES = 16
NUM_WORKERS = 32
CHUNK = 128
REL_ROWS = 64


def _vrsqrt(s, steps=3):
    i = lax.bitcast_convert_type(s, jnp.int32)
    i = jnp.int32(0x5F3759DF) - (i >> 1)
    y = lax.bitcast_convert_type(i, jnp.float32)
    h = s * 0.5
    for _ in range(steps):
        y = y * (1.5 - h * y * y)
    return y


def _sumsq_tree(vs):
    sq = [v * v for v in vs]
    while len(sq) > 1:
        sq = [a + b for a, b in zip(sq[::2], sq[1::2])]
    return sq[0]


def _row_scale(buf, r, nvec, steps=3):
    vs = [buf[r, pl.ds(k * LANES, LANES)] for k in range(nvec)]
    s = jnp.maximum(jnp.sum(_sumsq_tree(vs)), 1e-12)
    y = _vrsqrt(jnp.broadcast_to(s, (LANES,)), steps)
    return vs, y


def _transe_body(nch, nrel, ent_hbm, rel_hbm, idxe_hbm, idxr_hbm, out_hbm,
                 idxe_v, idxr_v, reln_sp, bufe0, bufe1, bufe2,
                 bufr0, bufr1, bufo0, bufo1,
                 seme0, seme1, seme2, semr0, semr1, semo0, semo1):
    d = ent_hbm.shape[1]
    nvec = d // LANES
    sid = lax.axis_index("s")
    wid = sid * 2 + lax.axis_index("c")
    base = wid * (nch * CHUNK)

    bufe = [bufe0, bufe1, bufe2]
    bufr = [bufr0, bufr1]
    bufo = [bufo0, bufo1]
    seme = [seme0, seme1, seme2]
    semr = [semr0, semr1]
    semo = [semo0, semo1]
    def ent_gather(j, slot):
        return pltpu.async_copy(ent_hbm.at[idxe_v.at[j]], bufe[slot],
                                seme[slot])

    pltpu.sync_copy(idxe_hbm.at[wid], idxe_v)
    pltpu.sync_copy(idxr_hbm.at[wid], idxr_v)

    entc = [ent_gather(0, 0), ent_gather(1, 1), None]

    start = jnp.minimum(sid * REL_ROWS, nrel - REL_ROWS)
    pltpu.sync_copy(rel_hbm.at[pl.ds(start, REL_ROWS)],
                    bufe2.at[pl.ds(0, REL_ROWS)])

    @pl.loop(0, REL_ROWS)
    def _(r):
        vs, y = _row_scale(bufe2, r, nvec)
        for k in range(nvec):
            bufe2[r, pl.ds(k * LANES, LANES)] = vs[k] * y

    pltpu.sync_copy(bufe2.at[pl.ds(0, REL_ROWS)],
                    reln_sp.at[pl.ds(start, REL_ROWS)])
    plsc.subcore_barrier()

    relc = [
        pltpu.async_copy(reln_sp.at[idxr_v.at[0]], bufr[0], semr[0]),
        pltpu.async_copy(reln_sp.at[idxr_v.at[1]], bufr[1], semr[1]),
    ]
    outc = [None, None]

    for j in range(nch):
        cur = j % 2
        ecur = j % 3
        entc[ecur].wait()
        if j + 2 < nch:
            entc[(j + 2) % 3] = ent_gather(j + 2, (j + 2) % 3)
        relc[cur].wait()
        if outc[cur] is not None:
            outc[cur].wait()
        be, br, bo = bufe[ecur], bufr[cur], bufo[cur]

        @pl.loop(0, CHUNK, step=2)
        def _(r0):
            for r in (r0, r0 + 1):
                evs, ye = _row_scale(be, r, nvec, steps=2)
                for k in range(nvec):
                    bo[r, pl.ds(k * LANES, LANES)] = (
                        evs[k] * ye + br[r, pl.ds(k * LANES, LANES)])

        outc[cur] = pltpu.async_copy(
            bo, out_hbm.at[pl.ds(base + j * CHUNK, CHUNK)], semo[cur])
        if j + 2 < nch:
            relc[cur] = pltpu.async_copy(
                reln_sp.at[idxr_v.at[j + 2]], br, semr[cur])

    for cp in outc:
        if cp is not None:
            cp.wait()


def kernel(batch_source, batch_r, entity_embeddings, relation_embeddings):
    b = batch_source.shape[0]
    d = entity_embeddings.shape[1]
    nrel = relation_embeddings.shape[0]
    nch = b // (NUM_WORKERS * CHUNK)
    idx_e = batch_source.astype(jnp.int32).reshape(NUM_WORKERS, nch, CHUNK)
    idx_r = batch_r.astype(jnp.int32).reshape(NUM_WORKERS, nch, CHUNK)

    mesh = plsc.VectorSubcoreMesh(core_axis_name="c", subcore_axis_name="s")
    cp = pltpu.CompilerParams()
    if "needs_layout_passes" in pltpu.CompilerParams.__dataclass_fields__:
        cp = dataclasses.replace(cp, needs_layout_passes=False)
    run = pl.kernel(
        functools.partial(_transe_body, nch, nrel),
        out_type=jax.ShapeDtypeStruct((b, d), jnp.float32),
        mesh=mesh,
        scratch_types=(
            [pltpu.VMEM((nch, CHUNK), jnp.int32)] * 2
            + [pltpu.VMEM_SHARED((nrel, d), jnp.float32)]
            + [pltpu.VMEM((CHUNK, d), jnp.float32)] * 7
            + [pltpu.SemaphoreType.DMA] * 7
        ),
        compiler_params=cp,
    )
    return run(entity_embeddings, relation_embeddings, idx_e, idx_r)


# speedup vs baseline: 1.4745x; 1.0118x over previous
import dataclasses
import functools

import jax
import jax.numpy as jnp
from jax import lax
from jax.experimental import pallas as pl
from jax.experimental.pallas import tpu as pltpu
from jax.experimental.pallas import tpu_sc as plsc

LANES = 16
NUM_WORKERS = 32
CHUNK = 128
REL_ROWS = 64


def _vrsqrt(s, steps=3):
    i = lax.bitcast_convert_type(s, jnp.int32)
    i = jnp.int32(0x5F3759DF) - (i >> 1)
    y = lax.bitcast_convert_type(i, jnp.float32)
    h = s * 0.5
    for _ in range(steps):
        y = y * (1.5 - h * y * y)
    return y


def _sumsq_tree(vs):
    sq = [v * v for v in vs]
    while len(sq) > 1:
        sq = [a + b for a, b in zip(sq[::2], sq[1::2])]
    return sq[0]


def _row_scale(buf, r, nvec, steps=3):
    vs = [buf[r, pl.ds(k * LANES, LANES)] for k in range(nvec)]
    s = jnp.maximum(jnp.sum(_sumsq_tree(vs)), 1e-12)
    y = _vrsqrt(jnp.broadcast_to(s, (LANES,)), steps)
    return vs, y


def _transe_body(nch, nrel, ent_hbm, rel_hbm, idxe_hbm, idxr_hbm, out_hbm,
                 idxe_v, idxr_v, reln_sp, bufe0, bufe1, bufe2,
                 bufr0, bufr1, bufo0, bufo1,
                 seme0, seme1, seme2, semr0, semr1, semo0, semo1):
    d = ent_hbm.shape[1]
    nvec = d // LANES
    sid = lax.axis_index("s")
    wid = sid * 2 + lax.axis_index("c")
    base = wid * (nch * CHUNK)

    bufe = [bufe0, bufe1, bufe2]
    bufr = [bufr0, bufr1]
    bufo = [bufo0, bufo1]
    seme = [seme0, seme1, seme2]
    semr = [semr0, semr1]
    semo = [semo0, semo1]
    def ent_gather(j, slot):
        return pltpu.async_copy(ent_hbm.at[idxe_v.at[j]], bufe[slot],
                                seme[slot])

    pltpu.sync_copy(idxe_hbm.at[wid], idxe_v)
    pltpu.sync_copy(idxr_hbm.at[wid], idxr_v)

    entc = [ent_gather(0, 0), ent_gather(1, 1), None]

    start = jnp.minimum(sid * REL_ROWS, nrel - REL_ROWS)
    pltpu.sync_copy(rel_hbm.at[pl.ds(start, REL_ROWS)],
                    bufe2.at[pl.ds(0, REL_ROWS)])

    @pl.loop(0, REL_ROWS)
    def _(r):
        vs, y = _row_scale(bufe2, r, nvec)
        for k in range(nvec):
            bufe2[r, pl.ds(k * LANES, LANES)] = vs[k] * y

    pltpu.sync_copy(bufe2.at[pl.ds(0, REL_ROWS)],
                    reln_sp.at[pl.ds(start, REL_ROWS)])
    plsc.subcore_barrier()

    relc = [
        pltpu.async_copy(reln_sp.at[idxr_v.at[0]], bufr[0], semr[0]),
        pltpu.async_copy(reln_sp.at[idxr_v.at[1]], bufr[1], semr[1]),
    ]
    outc = [None, None]

    for j in range(nch):
        cur = j % 2
        ecur = j % 3
        entc[ecur].wait()
        if j + 2 < nch:
            entc[(j + 2) % 3] = ent_gather(j + 2, (j + 2) % 3)
        relc[cur].wait()
        if outc[cur] is not None:
            outc[cur].wait()
        be, br, bo = bufe[ecur], bufr[cur], bufo[cur]

        @pl.loop(0, CHUNK)
        def _(r0):
            for r in (r0,):
                evs, ye = _row_scale(be, r, nvec, steps=2)
                for k in range(nvec):
                    bo[r, pl.ds(k * LANES, LANES)] = (
                        evs[k] * ye + br[r, pl.ds(k * LANES, LANES)])

        outc[cur] = pltpu.async_copy(
            bo, out_hbm.at[pl.ds(base + j * CHUNK, CHUNK)], semo[cur])
        if j + 2 < nch:
            relc[cur] = pltpu.async_copy(
                reln_sp.at[idxr_v.at[j + 2]], br, semr[cur])

    for cp in outc:
        if cp is not None:
            cp.wait()


def kernel(batch_source, batch_r, entity_embeddings, relation_embeddings):
    b = batch_source.shape[0]
    d = entity_embeddings.shape[1]
    nrel = relation_embeddings.shape[0]
    nch = b // (NUM_WORKERS * CHUNK)
    idx_e = batch_source.astype(jnp.int32).reshape(NUM_WORKERS, nch, CHUNK)
    idx_r = batch_r.astype(jnp.int32).reshape(NUM_WORKERS, nch, CHUNK)

    mesh = plsc.VectorSubcoreMesh(core_axis_name="c", subcore_axis_name="s")
    cp = pltpu.CompilerParams()
    if "needs_layout_passes" in pltpu.CompilerParams.__dataclass_fields__:
        cp = dataclasses.replace(cp, needs_layout_passes=False)
    run = pl.kernel(
        functools.partial(_transe_body, nch, nrel),
        out_type=jax.ShapeDtypeStruct((b, d), jnp.float32),
        mesh=mesh,
        scratch_types=(
            [pltpu.VMEM((nch, CHUNK), jnp.int32)] * 2
            + [pltpu.VMEM_SHARED((nrel, d), jnp.float32)]
            + [pltpu.VMEM((CHUNK, d), jnp.float32)] * 7
            + [pltpu.SemaphoreType.DMA] * 7
        ),
        compiler_params=cp,
    )
    return run(entity_embeddings, relation_embeddings, idx_e, idx_r)
